# baseline (speedup 1.0000x reference)
import functools

import jax
import jax.numpy as jnp
from jax import lax
from jax.experimental import pallas as pl
from jax.experimental.pallas import tpu as pltpu
from jax.experimental.pallas import tpu_sc as plsc

NUM_USER = 50000
NUM_ITEM = 50000
NUM_EDGE = 800000
EMBED_DIM = 64
HALF_DIM = 32
NUM_LAYER = 3
BATCH = 4096
N_NEG = 64

N_TILE = 16
N_CORE = 2
CHUNK = 128
BLK = 28
NBLK = 14
TILE_CHUNKS = BLK * NBLK
TILE_EDGES = TILE_CHUNKS * CHUNK
EDGE_PAD = N_TILE * TILE_EDGES
N_PAD = EDGE_PAD - NUM_EDGE
ACC_ROWS = 51200
ROWS_PER_TILE = ACC_ROWS // N_TILE
NRING = 4
NQ = 40
QROWS = ROWS_PER_TILE // NQ

_MESH = plsc.VectorSubcoreMesh(core_axis_name="c", subcore_axis_name="s")
_SC_PARAMS = pltpu.CompilerParams(use_tc_tiling_on_sc=False,
                                  needs_layout_passes=False)


def _deg_body(z1, du_idx, di_idx, deg_u, deg_i, acc, onesv, didxv, zstage):
    c = lax.axis_index("c")
    t = lax.axis_index("s")

    def fill_ones(i, _):
        onesv[pl.ds(i * 16, 16)] = jnp.ones((16,), jnp.float32)
        return _

    lax.fori_loop(0, CHUNK // 16, fill_ones, None)
    rpt = t * ROWS_PER_TILE
    pltpu.sync_copy(z1.at[pl.ds(rpt, ROWS_PER_TILE)], zstage)
    pltpu.sync_copy(zstage, acc.at[pl.ds(rpt, ROWS_PER_TILE)])
    plsc.subcore_barrier()

    def blk(b, _):
        crow = t * TILE_CHUNKS + b * BLK

        @pl.when(c == 0)
        def _():
            pltpu.sync_copy(du_idx.at[pl.ds(crow, BLK)], didxv)

        @pl.when(c == 1)
        def _():
            pltpu.sync_copy(di_idx.at[pl.ds(crow, BLK)], didxv)

        def chunk(j, _):
            pltpu.sync_copy(onesv, acc.at[didxv.at[j]], add=True)
            return _

        lax.fori_loop(0, BLK, chunk, None)
        return _

    lax.fori_loop(0, NBLK, blk, None)
    plsc.subcore_barrier()
    rb = t * ROWS_PER_TILE
    pltpu.sync_copy(acc.at[pl.ds(rb, ROWS_PER_TILE)], zstage)

    @pl.when(c == 0)
    def _():
        pltpu.sync_copy(zstage, deg_u.at[pl.ds(rb, ROWS_PER_TILE)])

    @pl.when(c == 1)
    def _():
        pltpu.sync_copy(zstage, deg_i.at[pl.ds(rb, ROWS_PER_TILE)])


_k_deg = pl.kernel(
    _deg_body,
    out_type=[jax.ShapeDtypeStruct((ACC_ROWS,), jnp.float32),
              jax.ShapeDtypeStruct((ACC_ROWS,), jnp.float32)],
    mesh=_MESH,
    compiler_params=_SC_PARAMS,
    scratch_types=[
        pltpu.VMEM_SHARED((ACC_ROWS,), jnp.float32),
        pltpu.VMEM((CHUNK,), jnp.float32),
        pltpu.VMEM((BLK, CHUNK), jnp.int32),
        pltpu.VMEM((ROWS_PER_TILE,), jnp.float32),
    ],
)


def _prop_body(s0, s1, z2, sidx, didx, inv, d0, d1, acc, gbuf, sidxv, didxv,
               stage, invv, *sems):
    c = lax.axis_index("c")
    t = lax.axis_index("s")

    pltpu.sync_copy(z2.at[pl.ds(t * ROWS_PER_TILE, QROWS)], stage)

    def zero_q(q, _):
        r = t * ROWS_PER_TILE + q * QROWS
        pltpu.sync_copy(stage, acc.at[pl.ds(r, QROWS)])
        return _

    lax.fori_loop(0, NQ, zero_q, None)
    plsc.subcore_barrier()

    def fire(j, b):
        islice = sidxv.at[pl.ds(j * CHUNK, CHUNK)]
        dst = gbuf.at[pl.ds(b * CHUNK, CHUNK)]

        @pl.when(c == 0)
        def _():
            pltpu.async_copy(s0.at[islice], dst, sems[b])

        @pl.when(c == 1)
        def _():
            pltpu.async_copy(s1.at[islice], dst, sems[b])

    def blk(b, _):
        eoff = t * TILE_EDGES + b * (BLK * CHUNK)
        pltpu.sync_copy(sidx.at[pl.ds(eoff, BLK * CHUNK)], sidxv)
        crow = t * TILE_CHUNKS + b * BLK
        pltpu.sync_copy(didx.at[pl.ds(crow, BLK)], didxv)

        for q in range(NRING):
            fire(q, q)

        def group(g, _):
            for q in range(NRING):
                j = g * NRING + q
                gb = gbuf.at[pl.ds(q * CHUNK, CHUNK)]
                pltpu.make_async_copy(s0.at[sidxv.at[pl.ds(0, CHUNK)]],
                                      gb, sems[q]).wait()
                pltpu.sync_copy(gb, acc.at[didxv.at[j]], add=True)

                @pl.when(g < BLK // NRING - 1)
                def _():
                    fire(j + NRING, q)

            return _

        lax.fori_loop(0, BLK // NRING, group, None)
        return _

    lax.fori_loop(0, NBLK, blk, None)
    plsc.subcore_barrier()
    lane = jnp.arange(16, dtype=jnp.int32)

    def wb_q(q, _):
        r = t * ROWS_PER_TILE + q * QROWS
        pltpu.sync_copy(acc.at[pl.ds(r, QROWS)], stage)
        pltpu.sync_copy(inv.at[pl.ds(r, QROWS)], invv)

        def scale_rg(rg, _):
            iv = invv[pl.ds(rg * 16, 16)]
            rows = rg * 16 + lane
            for col in range(HALF_DIM):
                cols = jnp.full((16,), col, dtype=jnp.int32)
                vals = plsc.load_gather(stage, [rows, cols])
                plsc.store_scatter(stage, [rows, cols], vals * iv)
            return _

        lax.fori_loop(0, QROWS // 16, scale_rg, None)

        @pl.when(c == 0)
        def _():
            pltpu.sync_copy(stage, d0.at[pl.ds(r, QROWS)])

        @pl.when(c == 1)
        def _():
            pltpu.sync_copy(stage, d1.at[pl.ds(r, QROWS)])

        return _

    lax.fori_loop(0, NQ, wb_q, None)


_k_prop = pl.kernel(
    _prop_body,
    out_type=[jax.ShapeDtypeStruct((ACC_ROWS, HALF_DIM), jnp.float32),
              jax.ShapeDtypeStruct((ACC_ROWS, HALF_DIM), jnp.float32)],
    mesh=_MESH,
    compiler_params=_SC_PARAMS,
    scratch_types=[
        pltpu.VMEM_SHARED((ACC_ROWS, HALF_DIM), jnp.float32),
        pltpu.VMEM((NRING * CHUNK, HALF_DIM), jnp.float32),
        pltpu.VMEM((BLK * CHUNK,), jnp.int32),
        pltpu.VMEM((BLK, CHUNK), jnp.int32),
        pltpu.VMEM((QROWS, HALF_DIM), jnp.float32),
        pltpu.VMEM((QROWS,), jnp.float32),
    ] + [pltpu.SemaphoreType.DMA] * NRING,
)


NEG_CHUNKS = BATCH * N_NEG // CHUNK
GGRP = 4
NEG_GROUPS = 16
NEG_PER_W = NEG_CHUNKS // (N_TILE * N_CORE)
B_CHUNKS = BATCH // CHUNK


def _gather_body(uemb, iemb, uidx, iidx, nidx, ue, pe, ne,
                 gbuf, uidxv, nidxv, *sems):
    c = lax.axis_index("c")
    s = lax.axis_index("s")
    w = s * N_CORE + c

    pltpu.sync_copy(uidx.at[pl.ds(w, 1)], uidxv)
    g0 = gbuf.at[pl.ds(0, CHUNK)]
    pltpu.async_copy(uemb.at[uidxv.at[0]], g0, sems[0]).wait()
    pltpu.sync_copy(g0, ue.at[pl.ds(w * CHUNK, CHUNK)])
    pltpu.sync_copy(iidx.at[pl.ds(w, 1)], uidxv)
    pltpu.async_copy(iemb.at[uidxv.at[0]], g0, sems[0]).wait()
    pltpu.sync_copy(g0, pe.at[pl.ds(w * CHUNK, CHUNK)])
    pltpu.sync_copy(nidx.at[pl.ds(w * NEG_PER_W, NEG_PER_W)], nidxv)

    def fire_group(g, h):
        for b in range(GGRP):
            pltpu.async_copy(
                iemb.at[nidxv.at[g * GGRP + b]],
                gbuf.at[pl.ds((h * GGRP + b) * CHUNK, CHUNK)],
                sems[h * GGRP + b])

    def wait_group(h):
        for b in range(GGRP):
            pltpu.make_async_copy(
                iemb.at[nidxv.at[pl.ds(0, CHUNK)]],
                gbuf.at[pl.ds((h * GGRP + b) * CHUNK, CHUNK)],
                sems[h * GGRP + b]).wait()

    fire_group(0, 0)

    def super_group(sg, _):
        for h in range(2):
            g = sg * 2 + h
            wait_group(h)

            @pl.when(g < NEG_GROUPS - 1)
            def _():
                fire_group(g + 1, 1 - h)

            pltpu.sync_copy(
                gbuf.at[pl.ds(h * GGRP * CHUNK, GGRP * CHUNK)],
                ne.at[pl.ds((w * NEG_PER_W + g * GGRP) * CHUNK,
                            GGRP * CHUNK)])
        return _

    lax.fori_loop(0, NEG_GROUPS // 2, super_group, None)


_k_gather = pl.kernel(
    _gather_body,
    out_type=[jax.ShapeDtypeStruct((BATCH, EMBED_DIM), jnp.float32),
              jax.ShapeDtypeStruct((BATCH, EMBED_DIM), jnp.float32),
              jax.ShapeDtypeStruct((BATCH * N_NEG, EMBED_DIM), jnp.float32)],
    mesh=_MESH,
    compiler_params=_SC_PARAMS,
    scratch_types=[
        pltpu.VMEM((2 * GGRP * CHUNK, EMBED_DIM), jnp.float32),
        pltpu.VMEM((1, CHUNK), jnp.int32),
        pltpu.VMEM((NEG_PER_W, CHUNK), jnp.int32),
    ] + [pltpu.SemaphoreType.DMA] * (2 * GGRP),
)


RB = 1600
N_RB = ACC_ROWS // RB


def _factors_body(du_ref, di_ref, iu_ref, ru_ref, fu_ref, ii_ref, ri_ref,
                  fi_ref):
    du = jnp.maximum(du_ref[...], 1.0)
    di = jnp.maximum(di_ref[...], 1.0)
    iu_ref[...] = 1.0 / du
    ru_ref[...] = lax.rsqrt(du)
    fu_ref[...] = 0.25 * jnp.sqrt(du)
    ii_ref[...] = 1.0 / di
    ri_ref[...] = lax.rsqrt(di)
    fi_ref[...] = 0.25 * jnp.sqrt(di)


def _factors(deg_u, deg_i):
    return pl.pallas_call(
        _factors_body,
        grid=(N_RB,),
        in_specs=[pl.BlockSpec((RB, 1), lambda b: (b, 0))] * 2,
        out_specs=[pl.BlockSpec((RB, 1), lambda b: (b, 0))] * 6,
        out_shape=[jax.ShapeDtypeStruct((ACC_ROWS, 1), jnp.float32)] * 6,
    )(deg_u.reshape(ACC_ROWS, 1), deg_i.reshape(ACC_ROWS, 1))


def _scale_split_body(ut_ref, it_ref, ru_ref, ri_ref,
                      u0_ref, u1_ref, i0_ref, i1_ref):
    hu = ut_ref[...] * ru_ref[...]
    hi = it_ref[...] * ri_ref[...]
    u0_ref[...] = hu[:, :HALF_DIM]
    u1_ref[...] = hu[:, HALF_DIM:]
    i0_ref[...] = hi[:, :HALF_DIM]
    i1_ref[...] = hi[:, HALF_DIM:]


def _scale_split(ut, it, ru, ri):
    return pl.pallas_call(
        _scale_split_body,
        grid=(N_RB,),
        in_specs=[pl.BlockSpec((RB, EMBED_DIM), lambda b: (b, 0))] * 2
                 + [pl.BlockSpec((RB, 1), lambda b: (b, 0))] * 2,
        out_specs=[pl.BlockSpec((RB, HALF_DIM), lambda b: (b, 0))] * 4,
        out_shape=[jax.ShapeDtypeStruct((ACC_ROWS, HALF_DIM), jnp.float32)] * 4,
    )(ut, it, ru, ri)


def _combine_body(*refs):
    (u00, u10, u20, u30, u01, u11, u21, u31, fu,
     i00, i10, i20, i30, i01, i11, i21, i31, fi, uo, io) = refs
    su = fu[...]
    si = fi[...]
    uh0 = (u00[...] + u10[...] + u20[...] + u30[...]) * su
    uh1 = (u01[...] + u11[...] + u21[...] + u31[...]) * su
    ih0 = (i00[...] + i10[...] + i20[...] + i30[...]) * si
    ih1 = (i01[...] + i11[...] + i21[...] + i31[...]) * si
    uo[...] = jnp.concatenate([uh0, uh1], axis=1)
    io[...] = jnp.concatenate([ih0, ih1], axis=1)


def _combine(uh0s, uh1s, fu, ih0s, ih1s, fi):
    return pl.pallas_call(
        _combine_body,
        grid=(N_RB,),
        in_specs=[pl.BlockSpec((RB, HALF_DIM), lambda b: (b, 0))] * 4
                 + [pl.BlockSpec((RB, HALF_DIM), lambda b: (b, 0))] * 4
                 + [pl.BlockSpec((RB, 1), lambda b: (b, 0))]
                 + [pl.BlockSpec((RB, HALF_DIM), lambda b: (b, 0))] * 4
                 + [pl.BlockSpec((RB, HALF_DIM), lambda b: (b, 0))] * 4
                 + [pl.BlockSpec((RB, 1), lambda b: (b, 0))],
        out_specs=[pl.BlockSpec((RB, EMBED_DIM), lambda b: (b, 0))] * 2,
        out_shape=[jax.ShapeDtypeStruct((ACC_ROWS, EMBED_DIM), jnp.float32)] * 2,
    )(*uh0s, *uh1s, fu, *ih0s, *ih1s, fi)


BB = 512


def _score_body(ue_ref, pe_ref, ne_ref, pos_ref, neg_ref, sq_ref):
    ue = ue_ref[...]
    pe = pe_ref[...]
    ne = ne_ref[...]
    pos_ref[...] = jnp.sum(ue * pe, axis=-1, keepdims=True)
    neg_ref[...] = lax.dot_general(
        ne, ue,
        dimension_numbers=(((2,), (1,)), ((0,), (0,))),
        preferred_element_type=jnp.float32,
    )
    v = jnp.sum(ue * ue) + jnp.sum(pe * pe) + jnp.sum(ne * ne)

    @pl.when(pl.program_id(0) == 0)
    def _init():
        sq_ref[...] = jnp.zeros((1, 128), dtype=jnp.float32)

    sq_ref[...] += jnp.full((1, 128), v / 128.0, dtype=jnp.float32)


def _score(ue, pe, ne):
    nblk = BATCH // BB
    return pl.pallas_call(
        _score_body,
        grid=(nblk,),
        in_specs=[
            pl.BlockSpec((BB, EMBED_DIM), lambda b: (b, 0)),
            pl.BlockSpec((BB, EMBED_DIM), lambda b: (b, 0)),
            pl.BlockSpec((BB, N_NEG, EMBED_DIM), lambda b: (b, 0, 0)),
        ],
        out_specs=[
            pl.BlockSpec((BB, 1), lambda b: (b, 0)),
            pl.BlockSpec((BB, N_NEG), lambda b: (b, 0)),
            pl.BlockSpec((1, 128), lambda b: (0, 0)),
        ],
        out_shape=[
            jax.ShapeDtypeStruct((BATCH, 1), jnp.float32),
            jax.ShapeDtypeStruct((BATCH, N_NEG), jnp.float32),
            jax.ShapeDtypeStruct((1, 128), jnp.float32),
        ],
    )(ue, pe, ne)


def kernel(user, item, item_negs, edge_u, edge_i, user_table, item_table):
    pad_ids = jnp.arange(N_PAD, dtype=jnp.int32)
    src_pad = (pad_ids * 97) % NUM_USER
    dst_pad = NUM_USER + pad_ids % (ACC_ROWS - NUM_USER)
    eu_src = jnp.concatenate([edge_u, src_pad])
    ei_src = jnp.concatenate([edge_i, src_pad])
    eu_dst = jnp.concatenate([edge_u, dst_pad]).reshape(EDGE_PAD // CHUNK, CHUNK)
    ei_dst = jnp.concatenate([edge_i, dst_pad]).reshape(EDGE_PAD // CHUNK, CHUNK)
    zeros1 = jnp.zeros((ACC_ROWS,), jnp.float32)
    zeros2 = jnp.zeros((ACC_ROWS, HALF_DIM), jnp.float32)

    deg_u, deg_i = _k_deg(zeros1, eu_dst, ei_dst)
    inv_u, ra_u, fs_u, inv_i, ra_i, fs_i = _factors(deg_u, deg_i)

    u_h0, u_h1 = [None] * 4, [None] * 4
    i_h0, i_h1 = [None] * 4, [None] * 4
    ut_pad = jnp.pad(user_table, ((0, ACC_ROWS - NUM_USER), (0, 0)))
    it_pad = jnp.pad(item_table, ((0, ACC_ROWS - NUM_ITEM), (0, 0)))
    u_h0[0], u_h1[0], i_h0[0], i_h1[0] = _scale_split(ut_pad, it_pad,
                                                      ra_u, ra_i)

    inv_u1 = inv_u.reshape(ACC_ROWS)
    inv_i1 = inv_i.reshape(ACC_ROWS)
    for k in range(NUM_LAYER):
        u_h0[k + 1], u_h1[k + 1] = _k_prop(i_h0[k], i_h1[k], zeros2,
                                           ei_src, eu_dst, inv_u1)
        i_h0[k + 1], i_h1[k + 1] = _k_prop(u_h0[k], u_h1[k], zeros2,
                                           eu_src, ei_dst, inv_i1)

    users_emb, items_emb = _combine(u_h0, u_h1, fs_u, i_h0, i_h1, fs_i)

    uidx = user.reshape(B_CHUNKS, CHUNK)
    iidx = item.reshape(B_CHUNKS, CHUNK)
    nidx = item_negs.reshape(NEG_CHUNKS, CHUNK)
    ue, pe, ne = _k_gather(users_emb, items_emb, uidx, iidx, nidx)

    pos, neg, sq = _score(ue, pe, ne.reshape(BATCH, N_NEG, EMBED_DIM))
    reg_loss = 0.5 * jnp.sum(sq) / float(BATCH)
    return pos, neg, reg_loss


# speedup vs baseline: 1.3251x; 1.3251x over previous
import functools

import jax
import jax.numpy as jnp
from jax import lax
from jax.experimental import pallas as pl
from jax.experimental.pallas import tpu as pltpu
from jax.experimental.pallas import tpu_sc as plsc

NUM_USER = 50000
NUM_ITEM = 50000
NUM_EDGE = 800000
EMBED_DIM = 64
HALF_DIM = 32
NUM_LAYER = 3
BATCH = 4096
N_NEG = 64

N_TILE = 16
N_CORE = 2
CHUNK = 128
BLK = 28
NBLK = 14
TILE_CHUNKS = BLK * NBLK
TILE_EDGES = TILE_CHUNKS * CHUNK
EDGE_PAD = N_TILE * TILE_EDGES
N_PAD = EDGE_PAD - NUM_EDGE
ACC_ROWS = 51200
ROWS_PER_TILE = ACC_ROWS // N_TILE
NRING = 4
NQ = 20
QROWS = ROWS_PER_TILE // NQ

_MESH = plsc.VectorSubcoreMesh(core_axis_name="c", subcore_axis_name="s")
_SC_PARAMS = pltpu.CompilerParams(use_tc_tiling_on_sc=False,
                                  needs_layout_passes=False)


def _deg_body(z1, du_idx, di_idx, deg_u, deg_i, acc, onesv, didxv, zstage):
    c = lax.axis_index("c")
    t = lax.axis_index("s")

    def fill_ones(i, _):
        onesv[pl.ds(i * 16, 16)] = jnp.ones((16,), jnp.float32)
        return _

    lax.fori_loop(0, CHUNK // 16, fill_ones, None)
    rpt = t * ROWS_PER_TILE
    pltpu.sync_copy(z1.at[pl.ds(rpt, ROWS_PER_TILE)], zstage)
    pltpu.sync_copy(zstage, acc.at[pl.ds(rpt, ROWS_PER_TILE)])
    plsc.subcore_barrier()

    def blk(b, _):
        crow = t * TILE_CHUNKS + b * BLK

        @pl.when(c == 0)
        def _():
            pltpu.sync_copy(du_idx.at[pl.ds(crow, BLK)], didxv)

        @pl.when(c == 1)
        def _():
            pltpu.sync_copy(di_idx.at[pl.ds(crow, BLK)], didxv)

        def chunk(j, _):
            pltpu.sync_copy(onesv, acc.at[didxv.at[j]], add=True)
            return _

        lax.fori_loop(0, BLK, chunk, None)
        return _

    lax.fori_loop(0, NBLK, blk, None)
    plsc.subcore_barrier()
    rb = t * ROWS_PER_TILE
    pltpu.sync_copy(acc.at[pl.ds(rb, ROWS_PER_TILE)], zstage)

    @pl.when(c == 0)
    def _():
        pltpu.sync_copy(zstage, deg_u.at[pl.ds(rb, ROWS_PER_TILE)])

    @pl.when(c == 1)
    def _():
        pltpu.sync_copy(zstage, deg_i.at[pl.ds(rb, ROWS_PER_TILE)])


_k_deg = pl.kernel(
    _deg_body,
    out_type=[jax.ShapeDtypeStruct((ACC_ROWS,), jnp.float32),
              jax.ShapeDtypeStruct((ACC_ROWS,), jnp.float32)],
    mesh=_MESH,
    compiler_params=_SC_PARAMS,
    scratch_types=[
        pltpu.VMEM_SHARED((ACC_ROWS,), jnp.float32),
        pltpu.VMEM((CHUNK,), jnp.float32),
        pltpu.VMEM((BLK, CHUNK), jnp.int32),
        pltpu.VMEM((ROWS_PER_TILE,), jnp.float32),
    ],
)


def _prop_body(s0, s1, z2, sidx, didx, d0, d1, acc, gbuf, sidxv, didxv,
               stage, *sems):
    c = lax.axis_index("c")
    t = lax.axis_index("s")

    pltpu.sync_copy(z2.at[pl.ds(t * ROWS_PER_TILE, QROWS)], stage)

    def zero_q(q, _):
        r = t * ROWS_PER_TILE + q * QROWS
        pltpu.sync_copy(stage, acc.at[pl.ds(r, QROWS)])
        return _

    lax.fori_loop(0, NQ, zero_q, None)
    plsc.subcore_barrier()

    def fire(j, b):
        islice = sidxv.at[pl.ds(j * CHUNK, CHUNK)]
        dst = gbuf.at[pl.ds(b * CHUNK, CHUNK)]

        @pl.when(c == 0)
        def _():
            pltpu.async_copy(s0.at[islice], dst, sems[b])

        @pl.when(c == 1)
        def _():
            pltpu.async_copy(s1.at[islice], dst, sems[b])

    def blk(b, _):
        eoff = t * TILE_EDGES + b * (BLK * CHUNK)
        pltpu.sync_copy(sidx.at[pl.ds(eoff, BLK * CHUNK)], sidxv)
        crow = t * TILE_CHUNKS + b * BLK
        pltpu.sync_copy(didx.at[pl.ds(crow, BLK)], didxv)

        for q in range(NRING):
            fire(q, q)

        def group(g, _):
            for q in range(NRING):
                j = g * NRING + q
                gb = gbuf.at[pl.ds(q * CHUNK, CHUNK)]
                pltpu.make_async_copy(s0.at[sidxv.at[pl.ds(0, CHUNK)]],
                                      gb, sems[q]).wait()
                pltpu.sync_copy(gb, acc.at[didxv.at[j]], add=True)

                @pl.when(g < BLK // NRING - 1)
                def _():
                    fire(j + NRING, q)

            return _

        lax.fori_loop(0, BLK // NRING, group, None)
        return _

    lax.fori_loop(0, NBLK, blk, None)
    plsc.subcore_barrier()

    def wb_q(q, _):
        r = t * ROWS_PER_TILE + q * QROWS
        pltpu.sync_copy(acc.at[pl.ds(r, QROWS)], stage)

        @pl.when(c == 0)
        def _():
            pltpu.sync_copy(stage, d0.at[pl.ds(r, QROWS)])

        @pl.when(c == 1)
        def _():
            pltpu.sync_copy(stage, d1.at[pl.ds(r, QROWS)])

        return _

    lax.fori_loop(0, NQ, wb_q, None)


_k_prop = pl.kernel(
    _prop_body,
    out_type=[jax.ShapeDtypeStruct((ACC_ROWS, HALF_DIM), jnp.float32),
              jax.ShapeDtypeStruct((ACC_ROWS, HALF_DIM), jnp.float32)],
    mesh=_MESH,
    compiler_params=_SC_PARAMS,
    scratch_types=[
        pltpu.VMEM_SHARED((ACC_ROWS, HALF_DIM), jnp.float32),
        pltpu.VMEM((NRING * CHUNK, HALF_DIM), jnp.float32),
        pltpu.VMEM((BLK * CHUNK,), jnp.int32),
        pltpu.VMEM((BLK, CHUNK), jnp.int32),
        pltpu.VMEM((QROWS, HALF_DIM), jnp.float32),
    ] + [pltpu.SemaphoreType.DMA] * NRING,
)


NEG_CHUNKS = BATCH * N_NEG // CHUNK
GGRP = 4
NEG_GROUPS = 16
NEG_PER_W = NEG_CHUNKS // (N_TILE * N_CORE)
B_CHUNKS = BATCH // CHUNK


def _gather_body(uemb, iemb, uidx, iidx, nidx, ue, pe, ne,
                 gbuf, uidxv, nidxv, *sems):
    c = lax.axis_index("c")
    s = lax.axis_index("s")
    w = s * N_CORE + c

    pltpu.sync_copy(uidx.at[pl.ds(w, 1)], uidxv)
    g0 = gbuf.at[pl.ds(0, CHUNK)]
    pltpu.async_copy(uemb.at[uidxv.at[0]], g0, sems[0]).wait()
    pltpu.sync_copy(g0, ue.at[pl.ds(w * CHUNK, CHUNK)])
    pltpu.sync_copy(iidx.at[pl.ds(w, 1)], uidxv)
    pltpu.async_copy(iemb.at[uidxv.at[0]], g0, sems[0]).wait()
    pltpu.sync_copy(g0, pe.at[pl.ds(w * CHUNK, CHUNK)])
    pltpu.sync_copy(nidx.at[pl.ds(w * NEG_PER_W, NEG_PER_W)], nidxv)

    def fire_group(g, h):
        for b in range(GGRP):
            pltpu.async_copy(
                iemb.at[nidxv.at[g * GGRP + b]],
                gbuf.at[pl.ds((h * GGRP + b) * CHUNK, CHUNK)],
                sems[h * GGRP + b])

    def wait_group(h):
        for b in range(GGRP):
            pltpu.make_async_copy(
                iemb.at[nidxv.at[pl.ds(0, CHUNK)]],
                gbuf.at[pl.ds((h * GGRP + b) * CHUNK, CHUNK)],
                sems[h * GGRP + b]).wait()

    fire_group(0, 0)

    def super_group(sg, _):
        for h in range(2):
            g = sg * 2 + h
            wait_group(h)

            @pl.when(g < NEG_GROUPS - 1)
            def _():
                fire_group(g + 1, 1 - h)

            pltpu.sync_copy(
                gbuf.at[pl.ds(h * GGRP * CHUNK, GGRP * CHUNK)],
                ne.at[pl.ds((w * NEG_PER_W + g * GGRP) * CHUNK,
                            GGRP * CHUNK)])
        return _

    lax.fori_loop(0, NEG_GROUPS // 2, super_group, None)


_k_gather = pl.kernel(
    _gather_body,
    out_type=[jax.ShapeDtypeStruct((BATCH, EMBED_DIM), jnp.float32),
              jax.ShapeDtypeStruct((BATCH, EMBED_DIM), jnp.float32),
              jax.ShapeDtypeStruct((BATCH * N_NEG, EMBED_DIM), jnp.float32)],
    mesh=_MESH,
    compiler_params=_SC_PARAMS,
    scratch_types=[
        pltpu.VMEM((2 * GGRP * CHUNK, EMBED_DIM), jnp.float32),
        pltpu.VMEM((1, CHUNK), jnp.int32),
        pltpu.VMEM((NEG_PER_W, CHUNK), jnp.int32),
    ] + [pltpu.SemaphoreType.DMA] * (2 * GGRP),
)


RB = 1600
N_RB = ACC_ROWS // RB


def _factors_body(du_ref, di_ref, iu_ref, ru_ref, fu_ref, ii_ref, ri_ref,
                  fi_ref):
    du = jnp.maximum(du_ref[...], 1.0)
    di = jnp.maximum(di_ref[...], 1.0)
    iu_ref[...] = 1.0 / du
    ru_ref[...] = lax.rsqrt(du)
    fu_ref[...] = 0.25 * jnp.sqrt(du)
    ii_ref[...] = 1.0 / di
    ri_ref[...] = lax.rsqrt(di)
    fi_ref[...] = 0.25 * jnp.sqrt(di)


def _factors(deg_u, deg_i):
    return pl.pallas_call(
        _factors_body,
        grid=(N_RB,),
        in_specs=[pl.BlockSpec((RB, 1), lambda b: (b, 0))] * 2,
        out_specs=[pl.BlockSpec((RB, 1), lambda b: (b, 0))] * 6,
        out_shape=[jax.ShapeDtypeStruct((ACC_ROWS, 1), jnp.float32)] * 6,
    )(deg_u.reshape(ACC_ROWS, 1), deg_i.reshape(ACC_ROWS, 1))


def _scale_split_body(ut_ref, it_ref, ru_ref, ri_ref,
                      u0_ref, u1_ref, i0_ref, i1_ref):
    hu = ut_ref[...] * ru_ref[...]
    hi = it_ref[...] * ri_ref[...]
    u0_ref[...] = hu[:, :HALF_DIM]
    u1_ref[...] = hu[:, HALF_DIM:]
    i0_ref[...] = hi[:, :HALF_DIM]
    i1_ref[...] = hi[:, HALF_DIM:]


def _scale_split(ut, it, ru, ri):
    return pl.pallas_call(
        _scale_split_body,
        grid=(N_RB,),
        in_specs=[pl.BlockSpec((RB, EMBED_DIM), lambda b: (b, 0))] * 2
                 + [pl.BlockSpec((RB, 1), lambda b: (b, 0))] * 2,
        out_specs=[pl.BlockSpec((RB, HALF_DIM), lambda b: (b, 0))] * 4,
        out_shape=[jax.ShapeDtypeStruct((ACC_ROWS, HALF_DIM), jnp.float32)] * 4,
    )(ut, it, ru, ri)


def _scale2_body(h0_ref, h1_ref, s_ref, o0_ref, o1_ref):
    s = s_ref[...]
    o0_ref[...] = h0_ref[...] * s
    o1_ref[...] = h1_ref[...] * s


def _scale2(h0, h1, s):
    return pl.pallas_call(
        _scale2_body,
        grid=(N_RB,),
        in_specs=[pl.BlockSpec((RB, HALF_DIM), lambda b: (b, 0))] * 2
                 + [pl.BlockSpec((RB, 1), lambda b: (b, 0))],
        out_specs=[pl.BlockSpec((RB, HALF_DIM), lambda b: (b, 0))] * 2,
        out_shape=[jax.ShapeDtypeStruct((ACC_ROWS, HALF_DIM), jnp.float32)] * 2,
    )(h0, h1, s)


def _combine_body(*refs):
    (u00, u10, u20, u30, u01, u11, u21, u31, fu,
     i00, i10, i20, i30, i01, i11, i21, i31, fi, uo, io) = refs
    su = fu[...]
    si = fi[...]
    uh0 = (u00[...] + u10[...] + u20[...] + u30[...]) * su
    uh1 = (u01[...] + u11[...] + u21[...] + u31[...]) * su
    ih0 = (i00[...] + i10[...] + i20[...] + i30[...]) * si
    ih1 = (i01[...] + i11[...] + i21[...] + i31[...]) * si
    uo[...] = jnp.concatenate([uh0, uh1], axis=1)
    io[...] = jnp.concatenate([ih0, ih1], axis=1)


def _combine(uh0s, uh1s, fu, ih0s, ih1s, fi):
    return pl.pallas_call(
        _combine_body,
        grid=(N_RB,),
        in_specs=[pl.BlockSpec((RB, HALF_DIM), lambda b: (b, 0))] * 4
                 + [pl.BlockSpec((RB, HALF_DIM), lambda b: (b, 0))] * 4
                 + [pl.BlockSpec((RB, 1), lambda b: (b, 0))]
                 + [pl.BlockSpec((RB, HALF_DIM), lambda b: (b, 0))] * 4
                 + [pl.BlockSpec((RB, HALF_DIM), lambda b: (b, 0))] * 4
                 + [pl.BlockSpec((RB, 1), lambda b: (b, 0))],
        out_specs=[pl.BlockSpec((RB, EMBED_DIM), lambda b: (b, 0))] * 2,
        out_shape=[jax.ShapeDtypeStruct((ACC_ROWS, EMBED_DIM), jnp.float32)] * 2,
    )(*uh0s, *uh1s, fu, *ih0s, *ih1s, fi)


BB = 512


def _score_body(ue_ref, pe_ref, ne_ref, pos_ref, neg_ref, sq_ref):
    ue = ue_ref[...]
    pe = pe_ref[...]
    ne = ne_ref[...]
    pos_ref[...] = jnp.sum(ue * pe, axis=-1, keepdims=True)
    neg_ref[...] = lax.dot_general(
        ne, ue,
        dimension_numbers=(((2,), (1,)), ((0,), (0,))),
        preferred_element_type=jnp.float32,
    )
    v = jnp.sum(ue * ue) + jnp.sum(pe * pe) + jnp.sum(ne * ne)

    @pl.when(pl.program_id(0) == 0)
    def _init():
        sq_ref[...] = jnp.zeros((1, 128), dtype=jnp.float32)

    sq_ref[...] += jnp.full((1, 128), v / 128.0, dtype=jnp.float32)


def _score(ue, pe, ne):
    nblk = BATCH // BB
    return pl.pallas_call(
        _score_body,
        grid=(nblk,),
        in_specs=[
            pl.BlockSpec((BB, EMBED_DIM), lambda b: (b, 0)),
            pl.BlockSpec((BB, EMBED_DIM), lambda b: (b, 0)),
            pl.BlockSpec((BB, N_NEG, EMBED_DIM), lambda b: (b, 0, 0)),
        ],
        out_specs=[
            pl.BlockSpec((BB, 1), lambda b: (b, 0)),
            pl.BlockSpec((BB, N_NEG), lambda b: (b, 0)),
            pl.BlockSpec((1, 128), lambda b: (0, 0)),
        ],
        out_shape=[
            jax.ShapeDtypeStruct((BATCH, 1), jnp.float32),
            jax.ShapeDtypeStruct((BATCH, N_NEG), jnp.float32),
            jax.ShapeDtypeStruct((1, 128), jnp.float32),
        ],
    )(ue, pe, ne)


def kernel(user, item, item_negs, edge_u, edge_i, user_table, item_table):
    pad_ids = jnp.arange(N_PAD, dtype=jnp.int32)
    src_pad = (pad_ids * 97) % NUM_USER
    dst_pad = NUM_USER + pad_ids % (ACC_ROWS - NUM_USER)
    eu_src = jnp.concatenate([edge_u, src_pad])
    ei_src = jnp.concatenate([edge_i, src_pad])
    eu_dst = jnp.concatenate([edge_u, dst_pad]).reshape(EDGE_PAD // CHUNK, CHUNK)
    ei_dst = jnp.concatenate([edge_i, dst_pad]).reshape(EDGE_PAD // CHUNK, CHUNK)
    zeros1 = jnp.zeros((ACC_ROWS,), jnp.float32)
    zeros2 = jnp.zeros((ACC_ROWS, HALF_DIM), jnp.float32)

    deg_u, deg_i = _k_deg(zeros1, eu_dst, ei_dst)
    inv_u, ra_u, fs_u, inv_i, ra_i, fs_i = _factors(deg_u, deg_i)

    u_h0, u_h1 = [None] * 4, [None] * 4
    i_h0, i_h1 = [None] * 4, [None] * 4
    ut_pad = jnp.pad(user_table, ((0, ACC_ROWS - NUM_USER), (0, 0)))
    it_pad = jnp.pad(item_table, ((0, ACC_ROWS - NUM_ITEM), (0, 0)))
    u_h0[0], u_h1[0], i_h0[0], i_h1[0] = _scale_split(ut_pad, it_pad,
                                                      ra_u, ra_i)

    for k in range(NUM_LAYER):
        a0, a1 = _k_prop(i_h0[k], i_h1[k], zeros2, ei_src, eu_dst)
        u_h0[k + 1], u_h1[k + 1] = _scale2(a0, a1, inv_u)
        b0, b1 = _k_prop(u_h0[k], u_h1[k], zeros2, eu_src, ei_dst)
        i_h0[k + 1], i_h1[k + 1] = _scale2(b0, b1, inv_i)

    users_emb, items_emb = _combine(u_h0, u_h1, fs_u, i_h0, i_h1, fs_i)

    uidx = user.reshape(B_CHUNKS, CHUNK)
    iidx = item.reshape(B_CHUNKS, CHUNK)
    nidx = item_negs.reshape(NEG_CHUNKS, CHUNK)
    ue, pe, ne = _k_gather(users_emb, items_emb, uidx, iidx, nidx)

    pos, neg, sq = _score(ue, pe, ne.reshape(BATCH, N_NEG, EMBED_DIM))
    reg_loss = 0.5 * jnp.sum(sq) / float(BATCH)
    return pos, neg, reg_loss
